# baseline (speedup 1.0000x reference)
import functools

import jax
import jax.numpy as jnp
from jax import lax
from jax.experimental import pallas as pl
from jax.experimental.pallas import tpu as pltpu
from jax.experimental.pallas import tpu_sc as plsc

VOCAB = 100000
EMBED = 128
BATCH = 1024
HIST = 200
TOTAL = BATCH * HIST

NC = 2
NS = 16
NW = NC * NS
B_PER_W = TOTAL // NW
CHUNK = 128
N_CHUNKS = B_PER_W // CHUNK
G = 2
ROWS_G = G * CHUNK
N_GROUPS = N_CHUNKS // G

_mesh = plsc.VectorSubcoreMesh(core_axis_name="c", subcore_axis_name="s")


@functools.partial(
    pl.kernel,
    mesh=_mesh,
    out_type=jax.ShapeDtypeStruct((TOTAL, EMBED), jnp.float32),
    scratch_types=[
        pltpu.VMEM((N_CHUNKS, CHUNK), jnp.int32),
        pltpu.VMEM((ROWS_G, EMBED), jnp.float32),
        pltpu.VMEM((ROWS_G, EMBED), jnp.float32),
        pltpu.VMEM((ROWS_G, EMBED), jnp.float32),
        pltpu.SemaphoreType.DMA,
        pltpu.SemaphoreType.DMA,
        pltpu.SemaphoreType.DMA,
        pltpu.SemaphoreType.DMA,
        pltpu.SemaphoreType.DMA,
        pltpu.SemaphoreType.DMA,
    ],
)
def _gather_kernel(idx_hbm, table_hbm, out_hbm, idx_v, rows_a, rows_b, rows_c,
                   sg_a, sg_b, sg_c, so_a, so_b, so_c):
    wid = lax.axis_index("s") * NC + lax.axis_index("c")
    base = wid * B_PER_W

    pltpu.sync_copy(idx_hbm.at[wid], idx_v)

    def fire_gathers(g, rows, sg):
        for j in range(G):
            pltpu.async_copy(table_hbm.at[idx_v.at[g * G + j]],
                             rows.at[pl.ds(j * CHUNK, CHUNK)], sg)

    def wait_gathers(rows, sg):
        pltpu.make_async_copy(table_hbm.at[pl.ds(0, ROWS_G)], rows, sg).wait()

    def out_slc(g):
        return out_hbm.at[pl.ds(base + g * ROWS_G, ROWS_G)]

    def step(g, cur, cur_sg, cur_so, oth, oth_sg, oth_so):
        wait_gathers(cur, cur_sg)
        pltpu.async_copy(cur, out_slc(g), cur_so)

        @pl.when(g > 0)
        def _():
            pltpu.make_async_copy(oth, out_slc(0), oth_so).wait()

        @pl.when(g < N_GROUPS - 2)
        def _():
            fire_gathers(g + 2, oth, oth_sg)

    fire_gathers(0, rows_a, sg_a)
    fire_gathers(1, rows_b, sg_b)

    def body(g, carry):
        r = g % 3

        @pl.when(r == 0)
        def _():
            step(g, rows_a, sg_a, so_a, rows_c, sg_c, so_c)

        @pl.when(r == 1)
        def _():
            step(g, rows_b, sg_b, so_b, rows_a, sg_a, so_a)

        @pl.when(r == 2)
        def _():
            step(g, rows_c, sg_c, so_c, rows_b, sg_b, so_b)

        return carry

    lax.fori_loop(0, N_GROUPS, body, 0)

    pltpu.make_async_copy(rows_a, out_slc(0), so_a).wait()


def kernel(input, table):
    idx = input.reshape(TOTAL).astype(jnp.int32)
    out = _gather_kernel(idx.reshape(NW, N_CHUNKS, CHUNK), table)
    return out.reshape(BATCH, HIST, EMBED)


# speedup vs baseline: 1.0191x; 1.0191x over previous
import functools

import jax
import jax.numpy as jnp
from jax import lax
from jax.experimental import pallas as pl
from jax.experimental.pallas import tpu as pltpu
from jax.experimental.pallas import tpu_sc as plsc

VOCAB = 100000
EMBED = 128
BATCH = 1024
HIST = 200
TOTAL = BATCH * HIST

NC = 2
NS = 16
NW = NC * NS
B_PER_W = TOTAL // NW
CHUNK = 128
N_GROUPS = B_PER_W // CHUNK
DEPTH = 6
LOOKAHEAD = 4

_mesh = plsc.VectorSubcoreMesh(core_axis_name="c", subcore_axis_name="s")


@functools.partial(
    pl.kernel,
    mesh=_mesh,
    out_type=jax.ShapeDtypeStruct((TOTAL, EMBED), jnp.float32),
    scratch_types=(
        [pltpu.VMEM((N_GROUPS, CHUNK), jnp.int32)]
        + [pltpu.VMEM((CHUNK, EMBED), jnp.float32)] * DEPTH
        + [pltpu.SemaphoreType.DMA] * (2 * DEPTH)
    ),
)
def _gather_kernel(idx_hbm, table_hbm, out_hbm, idx_v, *bufs_and_sems):
    bufs = bufs_and_sems[:DEPTH]
    sg = bufs_and_sems[DEPTH:2 * DEPTH]
    so = bufs_and_sems[2 * DEPTH:]

    wid = lax.axis_index("s") * NC + lax.axis_index("c")
    base = wid * B_PER_W

    pltpu.sync_copy(idx_hbm.at[wid], idx_v)

    def fire_gather(g, r):
        pltpu.async_copy(table_hbm.at[idx_v.at[g]], bufs[r], sg[r])

    def out_slc(g):
        return out_hbm.at[pl.ds(base + g * CHUNK, CHUNK)]

    def step(g, r):
        ra = (r + LOOKAHEAD) % DEPTH

        @pl.when(g >= DEPTH - LOOKAHEAD)
        def _():
            pltpu.make_async_copy(bufs[ra], out_slc(0), so[ra]).wait()

        @pl.when(g < N_GROUPS - LOOKAHEAD)
        def _():
            fire_gather(g + LOOKAHEAD, ra)

        pltpu.make_async_copy(table_hbm.at[pl.ds(0, CHUNK)],
                              bufs[r], sg[r]).wait()
        pltpu.async_copy(bufs[r], out_slc(g), so[r])

    for g in range(LOOKAHEAD):
        fire_gather(g, g)

    def body(g, carry):
        for r in range(DEPTH):
            @pl.when(g % DEPTH == r)
            def _(r=r):
                step(g, r)
        return carry

    lax.fori_loop(0, N_GROUPS, body, 0)

    for g in range(N_GROUPS - (DEPTH - LOOKAHEAD), N_GROUPS):
        r = g % DEPTH
        pltpu.make_async_copy(bufs[r], out_slc(0), so[r]).wait()


def kernel(input, table):
    idx = input.reshape(TOTAL).astype(jnp.int32)
    out = _gather_kernel(idx.reshape(NW, N_GROUPS, CHUNK), table)
    return out.reshape(BATCH, HIST, EMBED)
